# baseline (speedup 1.0000x reference)
import functools

import jax
import jax.numpy as jnp
from jax import lax
from jax.experimental import pallas as pl
from jax.experimental.pallas import tpu as pltpu
from jax.experimental.pallas import tpu_sc as plsc

D = 16
NC = 2
NS = 16
NW = NC * NS
CHUNK = 128
L = 16


def _tc_pack_body(eye, esel, t0, t1, t2, t3, o0, o1, o2, o3):
    for t, o in ((t0, o0), (t1, o1), (t2, o2), (t3, o3)):
        blk = t.shape[1]
        tmp = lax.dot_general(t[...], eye[...], (((0,), (0,)), ((), ())),
                              preferred_element_type=jnp.float32)
        g = tmp.reshape(blk // 8, 8, D)
        acc = jnp.dot(g[:, 0, :], esel[0], preferred_element_type=jnp.float32)
        for s in range(1, 8):
            acc += jnp.dot(g[:, s, :], esel[s],
                           preferred_element_type=jnp.float32)
        o[...] = acc


def _tc_pack_call(tables, eye, esel):
    N = tables[0].shape[1]
    BLK = 2048
    grid = ((N + BLK - 1) // BLK,)
    in_spec = pl.BlockSpec((D, BLK), lambda i: (0, i))
    out_spec = pl.BlockSpec((BLK * D // 128, 128), lambda i: (i, 0))
    full = lambda a: pl.BlockSpec(a.shape, lambda i: tuple(0 for _ in a.shape))
    return pl.pallas_call(
        _tc_pack_body,
        grid=grid,
        in_specs=[full(eye), full(esel)] + [in_spec] * 4,
        out_specs=[out_spec] * 4,
        out_shape=[jax.ShapeDtypeStruct((N * D // 128, 128), jnp.float32)] * 4,
    )(eye, esel, *tables)


def _sc_gather_call(uid, iid, f_ug, f_ig, f_um, f_im):
    B = uid.shape[0]
    ch = B // (NW * CHUNK)
    out_t = [jax.ShapeDtypeStruct((B, D), jnp.float32)] * 4
    mesh = plsc.VectorSubcoreMesh(core_axis_name="c", subcore_axis_name="s")

    @functools.partial(
        pl.kernel,
        out_type=out_t,
        mesh=mesh,
        scratch_types=[
            pltpu.VMEM((CHUNK,), jnp.int32),
            pltpu.VMEM((CHUNK,), jnp.int32),
            pltpu.VMEM((CHUNK * D,), jnp.float32),
            pltpu.VMEM((CHUNK * D,), jnp.float32),
            pltpu.VMEM((CHUNK * D,), jnp.float32),
            pltpu.VMEM((CHUNK * D,), jnp.float32),
            pltpu.VMEM((CHUNK, D), jnp.float32),
            pltpu.SemaphoreType.DMA,
        ],
    )
    def body(uid_h, iid_h, ug_h, ig_h, um_h, im_h,
             o_ug, o_ig, o_um, o_im,
             idx_u, idx_i, b_ug, b_ig, b_um, b_im, pack, sem):
        wid = lax.axis_index("s") * NC + lax.axis_index("c")
        for j in range(ch):
            base = (wid * ch + j) * CHUNK
            pltpu.sync_copy(uid_h.at[pl.ds(base, CHUNK)], idx_u)
            pltpu.sync_copy(iid_h.at[pl.ds(base, CHUNK)], idx_i)

            def fire(q, _):
                uv = idx_u[pl.ds(q * L, L)] * D
                iv = idx_i[pl.ds(q * L, L)] * D
                for k in range(L):
                    r = (q * L + k) * D
                    uo = pl.multiple_of(uv[k], 8)
                    io = pl.multiple_of(iv[k], 8)
                    pltpu.async_copy(ug_h.at[pl.ds(uo, D)], b_ug.at[pl.ds(r, D)], sem)
                    pltpu.async_copy(ig_h.at[pl.ds(io, D)], b_ig.at[pl.ds(r, D)], sem)
                    pltpu.async_copy(um_h.at[pl.ds(uo, D)], b_um.at[pl.ds(r, D)], sem)
                    pltpu.async_copy(im_h.at[pl.ds(io, D)], b_im.at[pl.ds(r, D)], sem)
                return _

            lax.fori_loop(0, CHUNK // L, fire, 0)

            def drain(r, _):
                for buf in (b_ug, b_ig, b_um, b_im):
                    pltpu.make_async_copy(
                        ug_h.at[pl.ds(0, D)], buf.at[pl.ds(r * D, D)], sem).wait()
                return _

            lax.fori_loop(0, CHUNK, drain, 0)

            for buf, out in ((b_ug, o_ug), (b_ig, o_ig),
                             (b_um, o_um), (b_im, o_im)):
                def repack(r, _, buf=buf):
                    pack[r, :] = buf[pl.ds(r * D, D)]
                    return _
                lax.fori_loop(0, CHUNK, repack, 0)
                pltpu.sync_copy(pack, out.at[pl.ds(base, CHUNK)])

    return body(uid, iid, f_ug, f_ig, f_um, f_im)


def _tc_dense_body(gu, gi, um, im, w1, b1, w2, b2, wl, bl, out):
    x_gmf = gu[...] * gi[...]
    h1 = jnp.dot(um[...], w1[:D], preferred_element_type=jnp.float32)
    h1 += jnp.dot(im[...], w1[D:], preferred_element_type=jnp.float32)
    h1 = jnp.maximum(h1 + b1[...], 0.0)
    h2 = jnp.dot(h1, w2[...], preferred_element_type=jnp.float32) + b2[...]
    h2 = jnp.maximum(h2, 0.0)
    o = jnp.dot(x_gmf, wl[:D], preferred_element_type=jnp.float32)
    o += jnp.dot(h2, wl[D:], preferred_element_type=jnp.float32)
    out[...] = o + bl[...]


def _tc_dense_call(gu, gi, um, im, W1, b1, W2, b2, WL, bL):
    B = gu.shape[0]
    BR = 2048
    grid = (B // BR,)
    row_spec = pl.BlockSpec((BR, D), lambda i: (i, 0))
    full = lambda a: pl.BlockSpec(a.shape, lambda i: tuple(0 for _ in a.shape))
    return pl.pallas_call(
        _tc_dense_body,
        grid=grid,
        in_specs=[row_spec, row_spec, row_spec, row_spec,
                  full(W1), full(b1), full(W2), full(b2), full(WL), full(bL)],
        out_specs=pl.BlockSpec((BR, 1), lambda i: (i, 0)),
        out_shape=jax.ShapeDtypeStruct((B, 1), jnp.float32),
    )(gu, gi, um, im, W1, b1, W2, b2, WL, bL)


def kernel(X, user_gmf, item_gmf, user_mlp, item_mlp, W1, b1, W2, b2, WL, bL):
    X = X.astype(jnp.int32)
    N = user_gmf.shape[0]
    eye = jnp.eye(D, dtype=jnp.float32)
    esel = (jnp.arange(128)[None, None, :]
            == (D * jnp.arange(8)[:, None, None]
                + jnp.arange(D)[None, :, None])).astype(jnp.float32)
    packed = _tc_pack_call((user_gmf.T, item_gmf.T, user_mlp.T, item_mlp.T),
                           eye, esel)
    flats = [p.reshape(N * D) for p in packed]
    ug, ig, um, im = _sc_gather_call(X[:, 0], X[:, 1], *flats)
    out = _tc_dense_call(
        ug, ig, um, im,
        W1, b1.reshape(1, D), W2, b2.reshape(1, D // 2),
        WL, bL.reshape(1, 1))
    return out


# speedup vs baseline: 3.7017x; 3.7017x over previous
import functools

import jax
import jax.numpy as jnp
from jax import lax
from jax.experimental import pallas as pl
from jax.experimental.pallas import tpu as pltpu
from jax.experimental.pallas import tpu_sc as plsc

D = 16
NT = 4
W = NT * D
NC = 2
NS = 16
NW = NC * NS
CHUNK = 128
L = 16


def _tc_pack_body(esel, tu, tum, ti, tim, o):
    x = jnp.concatenate([tu[...], tum[...], ti[...], tim[...]], axis=0)
    o[...] = lax.dot_general(x, esel[...], (((0,), (0,)), ((), ())),
                             preferred_element_type=jnp.float32)


def _tc_pack_call(tables, esel):
    N = tables[0].shape[1]
    BLK = 8192
    grid = ((N + BLK - 1) // BLK,)
    in_spec = pl.BlockSpec((D, BLK), lambda i: (0, i))
    full = lambda a: pl.BlockSpec(a.shape, lambda i: tuple(0 for _ in a.shape))
    return pl.pallas_call(
        _tc_pack_body,
        grid=grid,
        in_specs=[full(esel)] + [in_spec] * 4,
        out_specs=pl.BlockSpec((BLK, W), lambda i: (i, 0)),
        out_shape=jax.ShapeDtypeStruct((N, W), jnp.float32),
    )(esel, *tables)


def _sc_gather_call(uid, iid, packed):
    B = uid.shape[0]
    ch = B // (NW * CHUNK)
    mesh = plsc.VectorSubcoreMesh(core_axis_name="c", subcore_axis_name="s")

    @functools.partial(
        pl.kernel,
        out_type=[jax.ShapeDtypeStruct((B, W), jnp.float32)] * 2,
        mesh=mesh,
        scratch_types=[
            pltpu.VMEM((CHUNK,), jnp.int32),
            pltpu.VMEM((CHUNK,), jnp.int32),
            pltpu.VMEM((CHUNK, W), jnp.float32),
            pltpu.VMEM((CHUNK, W), jnp.float32),
            pltpu.SemaphoreType.DMA,
        ],
    )
    def body(uid_h, iid_h, p_h, o_u, o_i, idx_u, idx_i, b_u, b_i, sem):
        wid = lax.axis_index("s") * NC + lax.axis_index("c")
        for j in range(ch):
            base = (wid * ch + j) * CHUNK
            pltpu.sync_copy(uid_h.at[pl.ds(base, CHUNK)], idx_u)
            pltpu.sync_copy(iid_h.at[pl.ds(base, CHUNK)], idx_i)

            def fire(q, _):
                uv = idx_u[pl.ds(q * L, L)]
                iv = idx_i[pl.ds(q * L, L)]
                for k in range(L):
                    r = q * L + k
                    pltpu.async_copy(p_h.at[pl.ds(uv[k], 1)],
                                     b_u.at[pl.ds(r, 1)], sem)
                    pltpu.async_copy(p_h.at[pl.ds(iv[k], 1)],
                                     b_i.at[pl.ds(r, 1)], sem)
                return _

            lax.fori_loop(0, CHUNK // L, fire, 0)

            def drain(r, _):
                for buf in (b_u, b_i):
                    pltpu.make_async_copy(
                        p_h.at[pl.ds(0, 1)], buf.at[pl.ds(r, 1)], sem).wait()
                return _

            lax.fori_loop(0, CHUNK, drain, 0)
            pltpu.sync_copy(b_u, o_u.at[pl.ds(base, CHUNK)])
            pltpu.sync_copy(b_i, o_i.at[pl.ds(base, CHUNK)])

    return body(uid, iid, packed)


def _tc_dense_body(xu, xi, w1, b1, w2, b2, wl, bl, out):
    gu = xu[...][:, 0 * D:1 * D]
    um = xu[...][:, 1 * D:2 * D]
    gi = xi[...][:, 2 * D:3 * D]
    im = xi[...][:, 3 * D:4 * D]
    x_gmf = gu * gi
    h1 = jnp.dot(um, w1[:D], preferred_element_type=jnp.float32)
    h1 += jnp.dot(im, w1[D:], preferred_element_type=jnp.float32)
    h1 = jnp.maximum(h1 + b1[...], 0.0)
    h2 = jnp.dot(h1, w2[...], preferred_element_type=jnp.float32) + b2[...]
    h2 = jnp.maximum(h2, 0.0)
    o = jnp.dot(x_gmf, wl[:D], preferred_element_type=jnp.float32)
    o += jnp.dot(h2, wl[D:], preferred_element_type=jnp.float32)
    out[...] = o + bl[...]


def _tc_dense_call(xu, xi, W1, b1, W2, b2, WL, bL):
    B = xu.shape[0]
    BR = 2048
    grid = (B // BR,)
    row_spec = pl.BlockSpec((BR, W), lambda i: (i, 0))
    full = lambda a: pl.BlockSpec(a.shape, lambda i: tuple(0 for _ in a.shape))
    return pl.pallas_call(
        _tc_dense_body,
        grid=grid,
        in_specs=[row_spec, row_spec,
                  full(W1), full(b1), full(W2), full(b2), full(WL), full(bL)],
        out_specs=pl.BlockSpec((BR, 1), lambda i: (i, 0)),
        out_shape=jax.ShapeDtypeStruct((B, 1), jnp.float32),
    )(xu, xi, W1, b1, W2, b2, WL, bL)


def kernel(X, user_gmf, item_gmf, user_mlp, item_mlp, W1, b1, W2, b2, WL, bL):
    X = X.astype(jnp.int32)
    esel = jnp.eye(W, dtype=jnp.float32)
    uid, iid = X[:, 0], X[:, 1]
    packed = _tc_pack_call((user_gmf.T, user_mlp.T, item_gmf.T, item_mlp.T),
                           esel)
    xu, xi = _sc_gather_call(uid, iid, packed)
    out = _tc_dense_call(
        xu, xi,
        W1, b1.reshape(1, D), W2, b2.reshape(1, D // 2),
        WL, bL.reshape(1, 1))
    return out


# speedup vs baseline: 4.5687x; 1.2342x over previous
import functools

import jax
import jax.numpy as jnp
from jax import lax
from jax.experimental import pallas as pl
from jax.experimental.pallas import tpu as pltpu
from jax.experimental.pallas import tpu_sc as plsc

D = 16
NT = 4
W = NT * D
NC = 2
NS = 16
NW = NC * NS
CHUNK = 128
L = 16
BLK = 8192
NBLK = 62
H = NBLK * BLK


def _tc_pack_body(esel, tu, tum, ti, tim, thu, thum, thi, thim, o):
    lo = jnp.concatenate([tu[...], tum[...], ti[...], tim[...]], axis=0)
    hi = jnp.concatenate([thu[...], thum[...], thi[...], thim[...]], axis=0)
    e = esel[...]
    o[...] = (lax.dot_general(lo, e[:W], (((0,), (0,)), ((), ())),
                              preferred_element_type=jnp.float32)
              + lax.dot_general(hi, e[W:], (((0,), (0,)), ((), ())),
                                preferred_element_type=jnp.float32))


def _tc_pack_call(tables, esel):
    n_last = tables[0].shape[1] // BLK
    lo_spec = pl.BlockSpec((D, BLK), lambda i: (0, i))
    hi_spec = pl.BlockSpec((D, BLK),
                           lambda i: (0, jnp.minimum(i + NBLK, n_last)))
    full = lambda a: pl.BlockSpec(a.shape, lambda i: tuple(0 for _ in a.shape))
    return pl.pallas_call(
        _tc_pack_body,
        grid=(NBLK,),
        in_specs=[full(esel)] + [lo_spec] * 4 + [hi_spec] * 4,
        out_specs=pl.BlockSpec((BLK, 2 * W), lambda i: (i, 0)),
        out_shape=jax.ShapeDtypeStruct((H, 2 * W), jnp.float32),
    )(esel, *tables, *tables)


def _sc_gather_call(uid, iid, packed):
    B = uid.shape[0]
    ch = B // (NW * CHUNK)
    mesh = plsc.VectorSubcoreMesh(core_axis_name="c", subcore_axis_name="s")

    @functools.partial(
        pl.kernel,
        out_type=[jax.ShapeDtypeStruct((B, 2 * W), jnp.float32)] * 2,
        mesh=mesh,
        scratch_types=[
            pltpu.VMEM((CHUNK,), jnp.int32),
            pltpu.VMEM((CHUNK,), jnp.int32),
            pltpu.VMEM((CHUNK, 2 * W), jnp.float32),
            pltpu.VMEM((CHUNK, 2 * W), jnp.float32),
            pltpu.SemaphoreType.DMA,
        ],
    )
    def body(uid_h, iid_h, p_h, o_u, o_i, idx_u, idx_i, b_u, b_i, sem):
        wid = lax.axis_index("s") * NC + lax.axis_index("c")
        for j in range(ch):
            base = (wid * ch + j) * CHUNK
            pltpu.sync_copy(uid_h.at[pl.ds(base, CHUNK)], idx_u)
            pltpu.sync_copy(iid_h.at[pl.ds(base, CHUNK)], idx_i)

            def fire(q, _):
                uv = idx_u[pl.ds(q * L, L)]
                iv = idx_i[pl.ds(q * L, L)]
                uv = jnp.where(uv >= H, uv - H, uv)
                iv = jnp.where(iv >= H, iv - H, iv)
                for k in range(L):
                    r = q * L + k
                    pltpu.async_copy(p_h.at[pl.ds(uv[k], 1)],
                                     b_u.at[pl.ds(r, 1)], sem)
                    pltpu.async_copy(p_h.at[pl.ds(iv[k], 1)],
                                     b_i.at[pl.ds(r, 1)], sem)
                return _

            lax.fori_loop(0, CHUNK // L, fire, 0)

            def drain(r, _):
                for buf in (b_u, b_i):
                    pltpu.make_async_copy(
                        p_h.at[pl.ds(0, 1)], buf.at[pl.ds(r, 1)], sem).wait()
                return _

            lax.fori_loop(0, CHUNK, drain, 0)
            pltpu.sync_copy(b_u, o_u.at[pl.ds(base, CHUNK)])
            pltpu.sync_copy(b_i, o_i.at[pl.ds(base, CHUNK)])

    return body(uid, iid, packed)


def _tc_dense_body(xu, xi, u2, i2, w1, b1, w2, b2, wl, bl, out):
    su = u2[...] >= H
    si = i2[...] >= H
    xus = jnp.where(su, xu[...][:, W:], xu[...][:, :W])
    xis = jnp.where(si, xi[...][:, W:], xi[...][:, :W])
    gu = xus[:, 0 * D:1 * D]
    um = xus[:, 1 * D:2 * D]
    gi = xis[:, 2 * D:3 * D]
    im = xis[:, 3 * D:4 * D]
    x_gmf = gu * gi
    h1 = jnp.dot(um, w1[:D], preferred_element_type=jnp.float32)
    h1 += jnp.dot(im, w1[D:], preferred_element_type=jnp.float32)
    h1 = jnp.maximum(h1 + b1[...], 0.0)
    h2 = jnp.dot(h1, w2[...], preferred_element_type=jnp.float32) + b2[...]
    h2 = jnp.maximum(h2, 0.0)
    o = jnp.dot(x_gmf, wl[:D], preferred_element_type=jnp.float32)
    o += jnp.dot(h2, wl[D:], preferred_element_type=jnp.float32)
    out[...] = o + bl[...]


def _tc_dense_call(xu, xi, u2, i2, W1, b1, W2, b2, WL, bL):
    B = xu.shape[0]
    BR = 2048
    grid = (B // BR,)
    row_spec = pl.BlockSpec((BR, 2 * W), lambda i: (i, 0))
    id_spec = pl.BlockSpec((BR, 1), lambda i: (i, 0))
    full = lambda a: pl.BlockSpec(a.shape, lambda i: tuple(0 for _ in a.shape))
    return pl.pallas_call(
        _tc_dense_body,
        grid=grid,
        in_specs=[row_spec, row_spec, id_spec, id_spec,
                  full(W1), full(b1), full(W2), full(b2), full(WL), full(bL)],
        out_specs=pl.BlockSpec((BR, 1), lambda i: (i, 0)),
        out_shape=jax.ShapeDtypeStruct((B, 1), jnp.float32),
    )(xu, xi, u2, i2, W1, b1, W2, b2, WL, bL)


def kernel(X, user_gmf, item_gmf, user_mlp, item_mlp, W1, b1, W2, b2, WL, bL):
    X = X.astype(jnp.int32)
    esel = jnp.eye(W, 2 * W, dtype=jnp.float32)
    esel = jnp.concatenate([esel, jnp.roll(esel, W, axis=1)], axis=0)
    uid, iid = X[:, 0], X[:, 1]
    packed = _tc_pack_call((user_gmf.T, user_mlp.T, item_gmf.T, item_mlp.T),
                           esel)
    xu, xi = _sc_gather_call(uid, iid, packed)
    out = _tc_dense_call(
        xu, xi, X[:, 0:1], X[:, 1:2],
        W1, b1.reshape(1, D), W2, b2.reshape(1, D // 2),
        WL, bL.reshape(1, 1))
    return out


# speedup vs baseline: 4.9748x; 1.0889x over previous
import functools

import jax
import jax.numpy as jnp
from jax import lax
from jax.experimental import pallas as pl
from jax.experimental.pallas import tpu as pltpu
from jax.experimental.pallas import tpu_sc as plsc

D = 16
NT = 4
W = NT * D
NC = 2
NS = 16
NW = NC * NS
CHUNK = 128
L = 16
BLK = 16384
NBLK = 31
H = NBLK * BLK


def _tc_pack_body(esel, tu, tum, ti, tim, thu, thum, thi, thim, o):
    lo = jnp.concatenate([tu[...], tum[...], ti[...], tim[...]], axis=0)
    hi = jnp.concatenate([thu[...], thum[...], thi[...], thim[...]], axis=0)
    e = esel[...]
    o[...] = (lax.dot_general(lo, e[:W], (((0,), (0,)), ((), ())),
                              preferred_element_type=jnp.float32)
              + lax.dot_general(hi, e[W:], (((0,), (0,)), ((), ())),
                                preferred_element_type=jnp.float32))


def _tc_pack_call(tables, esel):
    n_last = tables[0].shape[1] // BLK
    lo_spec = pl.BlockSpec((D, BLK), lambda i: (0, i))
    hi_spec = pl.BlockSpec((D, BLK),
                           lambda i: (0, jnp.minimum(i + NBLK, n_last)))
    full = lambda a: pl.BlockSpec(a.shape, lambda i: tuple(0 for _ in a.shape))
    return pl.pallas_call(
        _tc_pack_body,
        grid=(NBLK,),
        in_specs=[full(esel)] + [lo_spec] * 4 + [hi_spec] * 4,
        out_specs=pl.BlockSpec((BLK, 2 * W), lambda i: (i, 0)),
        out_shape=jax.ShapeDtypeStruct((H, 2 * W), jnp.float32),
    )(esel, *tables, *tables)


def _sc_gather_call(uid, iid, packed):
    B = uid.shape[0]
    ch = B // (NW * CHUNK)
    mesh = plsc.VectorSubcoreMesh(core_axis_name="c", subcore_axis_name="s")

    @functools.partial(
        pl.kernel,
        out_type=[jax.ShapeDtypeStruct((B, 2 * W), jnp.float32)] * 2,
        mesh=mesh,
        scratch_types=[
            pltpu.VMEM((CHUNK,), jnp.int32),
            pltpu.VMEM((CHUNK,), jnp.int32),
            pltpu.VMEM((CHUNK, 2 * W), jnp.float32),
            pltpu.VMEM((CHUNK, 2 * W), jnp.float32),
            pltpu.SemaphoreType.DMA,
        ],
    )
    def body(uid_h, iid_h, p_h, o_u, o_i, idx_u, idx_i, b_u, b_i, sem):
        wid = lax.axis_index("s") * NC + lax.axis_index("c")
        for j in range(ch):
            base = (wid * ch + j) * CHUNK
            pltpu.sync_copy(uid_h.at[pl.ds(base, CHUNK)], idx_u)
            pltpu.sync_copy(iid_h.at[pl.ds(base, CHUNK)], idx_i)

            def fire(q, _):
                uv = idx_u[pl.ds(q * L, L)]
                iv = idx_i[pl.ds(q * L, L)]
                uv = jnp.where(uv >= H, uv - H, uv)
                iv = jnp.where(iv >= H, iv - H, iv)
                for k in range(L):
                    r = q * L + k
                    pltpu.async_copy(p_h.at[pl.ds(uv[k], 1)],
                                     b_u.at[pl.ds(r, 1)], sem)
                    pltpu.async_copy(p_h.at[pl.ds(iv[k], 1)],
                                     b_i.at[pl.ds(r, 1)], sem)
                return _

            lax.fori_loop(0, CHUNK // L, fire, 0)

            def drain(r, _):
                for buf in (b_u, b_i):
                    pltpu.make_async_copy(
                        p_h.at[pl.ds(0, 1)], buf.at[pl.ds(r, 1)], sem).wait()
                return _

            lax.fori_loop(0, CHUNK, drain, 0)
            pltpu.sync_copy(b_u, o_u.at[pl.ds(base, CHUNK)])
            pltpu.sync_copy(b_i, o_i.at[pl.ds(base, CHUNK)])

    return body(uid, iid, packed)


def _tc_dense_body(xu, xi, x, w1, b1, w2, b2, wl, bl, out):
    su = x[...][:, 0:1] >= H
    si = x[...][:, 1:2] >= H
    xus = jnp.where(su, xu[...][:, W:], xu[...][:, :W])
    xis = jnp.where(si, xi[...][:, W:], xi[...][:, :W])
    gu = xus[:, 0 * D:1 * D]
    um = xus[:, 1 * D:2 * D]
    gi = xis[:, 2 * D:3 * D]
    im = xis[:, 3 * D:4 * D]
    x_gmf = gu * gi
    h1 = jnp.dot(um, w1[:D], preferred_element_type=jnp.float32)
    h1 += jnp.dot(im, w1[D:], preferred_element_type=jnp.float32)
    h1 = jnp.maximum(h1 + b1[...], 0.0)
    h2 = jnp.dot(h1, w2[...], preferred_element_type=jnp.float32) + b2[...]
    h2 = jnp.maximum(h2, 0.0)
    o = jnp.dot(x_gmf, wl[:D], preferred_element_type=jnp.float32)
    o += jnp.dot(h2, wl[D:], preferred_element_type=jnp.float32)
    out[...] = o + bl[...]


def _tc_dense_call(xu, xi, X, W1, b1, W2, b2, WL, bL):
    B = xu.shape[0]
    BR = 2048
    grid = (B // BR,)
    row_spec = pl.BlockSpec((BR, 2 * W), lambda i: (i, 0))
    id_spec = pl.BlockSpec((BR, 2), lambda i: (i, 0))
    full = lambda a: pl.BlockSpec(a.shape, lambda i: tuple(0 for _ in a.shape))
    return pl.pallas_call(
        _tc_dense_body,
        grid=grid,
        in_specs=[row_spec, row_spec, id_spec,
                  full(W1), full(b1), full(W2), full(b2), full(WL), full(bL)],
        out_specs=pl.BlockSpec((BR, 1), lambda i: (i, 0)),
        out_shape=jax.ShapeDtypeStruct((B, 1), jnp.float32),
    )(xu, xi, X, W1, b1, W2, b2, WL, bL)


def kernel(X, user_gmf, item_gmf, user_mlp, item_mlp, W1, b1, W2, b2, WL, bL):
    X = X.astype(jnp.int32)
    esel = jnp.eye(W, 2 * W, dtype=jnp.float32)
    esel = jnp.concatenate([esel, jnp.roll(esel, W, axis=1)], axis=0)
    uid, iid = X[:, 0], X[:, 1]
    packed = _tc_pack_call((user_gmf.T, user_mlp.T, item_gmf.T, item_mlp.T),
                           esel)
    xu, xi = _sc_gather_call(uid, iid, packed)
    out = _tc_dense_call(
        xu, xi, X,
        W1, b1.reshape(1, D), W2, b2.reshape(1, D // 2),
        WL, bL.reshape(1, 1))
    return out
